# baseline (device time: 707424 ns/iter reference)
import jax
import jax.numpy as jnp
from jax import lax
from jax.experimental import pallas as pl
from jax.experimental.pallas import tpu as pltpu

N_DEV = 4
S_LOC = 2048
D = 1024
HL = 8
DH = 128
SKV = 2048
Q_TILE = 256
N_TILES = S_LOC // Q_TILE
SCALE = 0.08838834764831843
NEG = -1e9

_CompilerParams = getattr(pltpu, "CompilerParams", None) or pltpu.TPUCompilerParams


def kernel(x, Wq, K_ext, V_ext, Wo):
    j = lax.axis_index("i")
    xb = x[0].astype(jnp.bfloat16)
    wq = Wq.astype(jnp.bfloat16)
    wo = Wo.astype(jnp.bfloat16)
    k_loc = lax.dynamic_slice_in_dim(K_ext[0], j * HL, HL, axis=1)
    v_loc = lax.dynamic_slice_in_dim(V_ext[0], j * HL, HL, axis=1)
    kt = jnp.transpose(k_loc, (1, 2, 0)).astype(jnp.bfloat16)
    vt = jnp.transpose(v_loc, (1, 0, 2)).astype(jnp.bfloat16)

    def body(x_ref, wq_ref, kt_ref, vt_ref, wo_ref, out_ref,
             xg_ref, rs_ref, ag_send, ag_recv, rs_send, rs_recv):
        my = lax.axis_index("i")
        right = lax.rem(my + 1, N_DEV)
        left = lax.rem(my + N_DEV - 1, N_DEV)

        barrier_sem = pltpu.get_barrier_semaphore()
        for nbr in (left, right):
            pl.semaphore_signal(barrier_sem, inc=1, device_id=(nbr,),
                                device_id_type=pl.DeviceIdType.MESH)
        pl.semaphore_wait(barrier_sem, 2)

        def p_tile(loader, row0, c):
            x_t = loader(row0)
            q = jnp.dot(x_t, wq_ref[...],
                        preferred_element_type=jnp.float32).astype(jnp.bfloat16)
            qi = lax.broadcasted_iota(jnp.int32, (Q_TILE, SKV), 0)
            ki = lax.broadcasted_iota(jnp.int32, (Q_TILE, SKV), 1)
            qb = (c * S_LOC + row0 + qi) // 64
            kb = ki // 64
            mask = (qb == kb) | (kb == 0) | ((qb + kb) % 3 == 0)
            parts = []
            for h in range(HL):
                qh = q[:, h * DH:(h + 1) * DH]
                s = jnp.dot(qh, kt_ref[h],
                            preferred_element_type=jnp.float32) * SCALE
                s = jnp.where(mask, s, NEG)
                m = jnp.max(s, axis=1, keepdims=True)
                w = jnp.exp(s - m)
                w = (w / jnp.sum(w, axis=1, keepdims=True)).astype(jnp.bfloat16)
                parts.append(jnp.dot(w, vt_ref[h],
                                     preferred_element_type=jnp.float32))
            ctx = jnp.concatenate(parts, axis=1).astype(jnp.bfloat16)
            return jnp.dot(ctx, wo_ref[...], preferred_element_type=jnp.float32)

        def chunk_loop(loader, c, write):
            def tbody(t, _):
                row0 = t * Q_TILE
                write(row0, p_tile(loader, row0, c))
                return 0
            lax.fori_loop(0, N_TILES, tbody, 0)

        for h in range(N_DEV - 1):
            src = x_ref if h == 0 else xg_ref.at[h - 1]
            rdma = pltpu.make_async_remote_copy(
                src_ref=src,
                dst_ref=xg_ref.at[h],
                send_sem=ag_send.at[h],
                recv_sem=ag_recv.at[h],
                device_id=(right,),
                device_id_type=pl.DeviceIdType.MESH,
            )
            rdma.start()
            rdma.wait()

        c0 = lax.rem(my - 1 + N_DEV, N_DEV)
        chunk_loop(lambda r0: xg_ref[0, pl.ds(r0, Q_TILE), :], c0,
                   lambda r0, P: rs_ref.__setitem__(
                       (0, pl.ds(r0, Q_TILE), slice(None)),
                       P.astype(jnp.bfloat16)))

        for s in range(N_DEV - 1):
            last = s == N_DEV - 2
            rdma = pltpu.make_async_remote_copy(
                src_ref=rs_ref.at[s],
                dst_ref=out_ref if last else rs_ref.at[s + 1],
                send_sem=rs_send.at[s],
                recv_sem=rs_recv.at[s],
                device_id=(right,),
                device_id_type=pl.DeviceIdType.MESH,
            )
            rdma.start()
            rdma.wait()
            cs = lax.rem(my - 2 - s + 2 * N_DEV, N_DEV)
            if not last:
                def acc_write(r0, P, _s=s):
                    acc = rs_ref[_s + 1, pl.ds(r0, Q_TILE), :].astype(jnp.float32)
                    rs_ref[_s + 1, pl.ds(r0, Q_TILE), :] = \
                        (acc + P).astype(jnp.bfloat16)
                chunk_loop(
                    lambda r0, _s=s: xg_ref[_s + 1, pl.ds(r0, Q_TILE), :],
                    cs, acc_write)
            else:
                def fin_write(r0, P):
                    acc = out_ref[pl.ds(r0, Q_TILE), :].astype(jnp.float32)
                    out_ref[pl.ds(r0, Q_TILE), :] = (acc + P).astype(jnp.bfloat16)
                chunk_loop(lambda r0: x_ref[pl.ds(r0, Q_TILE), :], cs, fin_write)

    out = pl.pallas_call(
        body,
        out_shape=jax.ShapeDtypeStruct((S_LOC, D), jnp.bfloat16),
        in_specs=[pl.BlockSpec(memory_space=pltpu.VMEM)] * 5,
        out_specs=pl.BlockSpec(memory_space=pltpu.VMEM),
        scratch_shapes=[
            pltpu.VMEM((N_DEV - 1, S_LOC, D), jnp.bfloat16),
            pltpu.VMEM((N_DEV - 1, S_LOC, D), jnp.bfloat16),
            pltpu.SemaphoreType.DMA((N_DEV - 1,)),
            pltpu.SemaphoreType.DMA((N_DEV - 1,)),
            pltpu.SemaphoreType.DMA((N_DEV - 1,)),
            pltpu.SemaphoreType.DMA((N_DEV - 1,)),
        ],
        compiler_params=_CompilerParams(collective_id=0),
    )(xb, wq, kt, vt, wo)
    return out.astype(jnp.float32)[None]


# device time: 415725 ns/iter; 1.7017x vs baseline; 1.7017x over previous
import jax
import jax.numpy as jnp
from jax import lax
from jax.experimental import pallas as pl
from jax.experimental.pallas import tpu as pltpu

N_DEV = 4
S_LOC = 2048
D = 1024
HL = 8
DH = 128
SKV = 2048
Q_TILE = 256
N_TILES = S_LOC // Q_TILE
SCALE = 0.08838834764831843
NEG = -1e9

_CompilerParams = getattr(pltpu, "CompilerParams", None) or pltpu.TPUCompilerParams


def kernel(x, Wq, K_ext, V_ext, Wo):
    j = lax.axis_index("i")
    xb = x[0].astype(jnp.bfloat16)
    wq = Wq.astype(jnp.bfloat16)
    wo = Wo.astype(jnp.bfloat16)
    k_loc = lax.dynamic_slice_in_dim(K_ext[0], j * HL, HL, axis=1)
    v_loc = lax.dynamic_slice_in_dim(V_ext[0], j * HL, HL, axis=1)
    kt = jnp.transpose(k_loc, (1, 2, 0)).astype(jnp.bfloat16)
    vt = jnp.transpose(v_loc, (1, 0, 2)).astype(jnp.bfloat16)

    def body(x_ref, wq_ref, kt_ref, vt_ref, wo_ref, out_ref,
             xg_ref, rs_ref, pmy_ref, tmp_ref,
             ag_send, ag_recv, rs_send, rs_recv):
        my = lax.axis_index("i")
        right = lax.rem(my + 1, N_DEV)
        left = lax.rem(my + N_DEV - 1, N_DEV)

        barrier_sem = pltpu.get_barrier_semaphore()
        for nbr in (left, right):
            pl.semaphore_signal(barrier_sem, inc=1, device_id=(nbr,),
                                device_id_type=pl.DeviceIdType.MESH)
        pl.semaphore_wait(barrier_sem, 2)

        def p_tile(loader, row0, c):
            x_t = loader(row0)
            q = jnp.dot(x_t, wq_ref[...],
                        preferred_element_type=jnp.float32).astype(jnp.bfloat16)
            qi = lax.broadcasted_iota(jnp.int32, (Q_TILE, SKV), 0)
            ki = lax.broadcasted_iota(jnp.int32, (Q_TILE, SKV), 1)
            qb = (c * S_LOC + row0 + qi) // 64
            kb = ki // 64
            mask = (qb == kb) | (kb == 0) | ((qb + kb) % 3 == 0)
            bias = jnp.where(mask, 0.0, NEG).astype(jnp.float32)
            parts = []
            for h in range(HL):
                qh = q[:, h * DH:(h + 1) * DH]
                s = jnp.dot(qh, kt_ref[h],
                            preferred_element_type=jnp.float32) * SCALE + bias
                w = jnp.exp(s)
                w = (w * (1.0 / jnp.sum(w, axis=1, keepdims=True))
                     ).astype(jnp.bfloat16)
                parts.append(jnp.dot(w, vt_ref[h],
                                     preferred_element_type=jnp.float32))
            ctx = jnp.concatenate(parts, axis=1).astype(jnp.bfloat16)
            return jnp.dot(ctx, wo_ref[...], preferred_element_type=jnp.float32)

        def chunk_into(loader, c, dst_ref):
            def tbody(t, _):
                row0 = t * Q_TILE
                dst_ref[pl.ds(row0, Q_TILE), :] = \
                    p_tile(loader, row0, c).astype(jnp.bfloat16)
                return 0
            lax.fori_loop(0, N_TILES, tbody, 0)

        def ag_rdma(h):
            src = x_ref if h == 0 else xg_ref.at[h - 1]
            return pltpu.make_async_remote_copy(
                src_ref=src, dst_ref=xg_ref.at[h],
                send_sem=ag_send.at[h], recv_sem=ag_recv.at[h],
                device_id=(right,), device_id_type=pl.DeviceIdType.MESH)

        def rs_rdma(s):
            dst = out_ref if s == N_DEV - 2 else rs_ref.at[s + 1]
            return pltpu.make_async_remote_copy(
                src_ref=rs_ref.at[s], dst_ref=dst,
                send_sem=rs_send.at[s], recv_sem=rs_recv.at[s],
                device_id=(right,), device_id_type=pl.DeviceIdType.MESH)

        ag = [ag_rdma(h) for h in range(N_DEV - 1)]
        rs = [rs_rdma(s) for s in range(N_DEV - 1)]

        c = [lax.rem(my - 1 - s + 2 * N_DEV, N_DEV) for s in range(N_DEV - 1)]

        ag[0].start()
        chunk_into(lambda r0: x_ref[pl.ds(r0, Q_TILE), :], my, pmy_ref)

        ag[0].wait_recv()
        ag[1].start()
        chunk_into(lambda r0: xg_ref[0, pl.ds(r0, Q_TILE), :], c[0], rs_ref.at[0])
        rs[0].start()

        ag[1].wait_recv()
        ag[2].start()
        chunk_into(lambda r0: xg_ref[1, pl.ds(r0, Q_TILE), :], c[1], tmp_ref)
        rs[0].wait_recv()
        rs_ref[1] = (rs_ref[1].astype(jnp.float32)
                     + tmp_ref[...].astype(jnp.float32)).astype(jnp.bfloat16)
        rs[1].start()

        ag[2].wait_recv()
        chunk_into(lambda r0: xg_ref[2, pl.ds(r0, Q_TILE), :], c[2], tmp_ref)
        rs[1].wait_recv()
        rs_ref[2] = (rs_ref[2].astype(jnp.float32)
                     + tmp_ref[...].astype(jnp.float32)).astype(jnp.bfloat16)
        rs[2].start()

        rs[2].wait_recv()
        out_ref[...] = (out_ref[...].astype(jnp.float32)
                        + pmy_ref[...].astype(jnp.float32)).astype(jnp.bfloat16)

        for r in ag + rs:
            r.wait_send()

    out = pl.pallas_call(
        body,
        out_shape=jax.ShapeDtypeStruct((S_LOC, D), jnp.bfloat16),
        in_specs=[pl.BlockSpec(memory_space=pltpu.VMEM)] * 5,
        out_specs=pl.BlockSpec(memory_space=pltpu.VMEM),
        scratch_shapes=[
            pltpu.VMEM((N_DEV - 1, S_LOC, D), jnp.bfloat16),
            pltpu.VMEM((N_DEV - 1, S_LOC, D), jnp.bfloat16),
            pltpu.VMEM((S_LOC, D), jnp.bfloat16),
            pltpu.VMEM((S_LOC, D), jnp.bfloat16),
            pltpu.SemaphoreType.DMA((N_DEV - 1,)),
            pltpu.SemaphoreType.DMA((N_DEV - 1,)),
            pltpu.SemaphoreType.DMA((N_DEV - 1,)),
            pltpu.SemaphoreType.DMA((N_DEV - 1,)),
        ],
        compiler_params=_CompilerParams(collective_id=0,
                                        vmem_limit_bytes=60 * 1024 * 1024),
    )(xb, wq, kt, vt, wo)
    return out.astype(jnp.float32)[None]
